# baseline (speedup 1.0000x reference)
import functools

import jax
import jax.numpy as jnp
from jax import lax
from jax.experimental import pallas as pl
from jax.experimental.pallas import tpu as pltpu
from jax.experimental.pallas import tpu_sc as plsc

_NUM_CODES = 1024
_HALF = 64
_COLORS = 16
_CC = 0.25

_T = 4096
_NW = 32
_CHUNK = 128


def _rownorm(w):
    return w / jnp.maximum(
        jnp.sqrt(jnp.sum(w * w, axis=1, keepdims=True)), 1e-12)


def _prep_body(ws_ref, wc_ref, wsn_ref, wcn_ref, bs_ref, bc_ref, tab_ref):
    wsn = _rownorm(ws_ref[...])
    wcn = _rownorm(wc_ref[...])
    wsn_ref[...] = wsn
    wcn_ref[...] = wcn

    io_s = jax.lax.broadcasted_iota(jnp.int32, (_NUM_CODES, 1), 0)
    hi = (io_s >> 6).astype(jnp.bfloat16)
    lo = (io_s & 63).astype(jnp.bfloat16)
    bs_ref[...] = jnp.concatenate(
        [hi, lo, jnp.zeros((_NUM_CODES, 126), jnp.bfloat16)], axis=1)

    io_c = jax.lax.broadcasted_iota(jnp.int32, (_COLORS, 1), 0)
    bc_ref[...] = jnp.concatenate(
        [io_c.astype(jnp.bfloat16), jnp.zeros((_COLORS, 127), jnp.bfloat16)],
        axis=1)

    tab_ref[...] = jnp.concatenate([
        jnp.broadcast_to(wsn[:, None, :], (_NUM_CODES, _COLORS, _HALF)),
        jnp.broadcast_to(wcn[None, :, :], (_NUM_CODES, _COLORS, _HALF)),
    ], axis=2)


def _tc_body(z_ref, wsn_ref, wcn_ref, bs_ref, bc_ref,
             cat_ref, si_ref, ci_ref, loss_ref):
    z = z_ref[...]
    zs = z[:, :_HALF]
    zc = z[:, _HALF:]
    nzs2 = jnp.sum(zs * zs, axis=1, keepdims=True)
    nzc2 = jnp.sum(zc * zc, axis=1, keepdims=True)
    nzs = jnp.maximum(jnp.sqrt(nzs2), 1e-12)
    nzc = jnp.maximum(jnp.sqrt(nzc2), 1e-12)
    zsn = zs * (1.0 / nzs)
    zcn = zc * (1.0 / nzc)

    ls = jnp.dot(zsn, wsn_ref[...].T,
                 preferred_element_type=jnp.float32)
    lc = jnp.dot(zcn, wcn_ref[...].T,
                 preferred_element_type=jnp.float32)

    mxs = jnp.max(ls, axis=1, keepdims=True)
    mxc = jnp.max(lc, axis=1, keepdims=True)
    ms = (ls == mxs).astype(jnp.bfloat16)
    mc = (lc == mxc).astype(jnp.bfloat16)

    rs = jnp.dot(ms, bs_ref[...], preferred_element_type=jnp.float32)
    rc = jnp.dot(mc, bc_ref[...], preferred_element_type=jnp.float32)

    si = (rs[:, 0:1] * 64.0 + rs[:, 1:2]).astype(jnp.int32)
    ci = rc[:, 0:1].astype(jnp.int32)
    si_ref[...] = si
    ci_ref[...] = ci
    cat_ref[...] = jnp.minimum(si * _COLORS + ci,
                               _NUM_CODES * _COLORS - 1)

    part = (jnp.sum(nzs2 + nzc2 + 2.0)
            - 2.0 * jnp.sum(mxs * nzs + mxc * nzc))
    loss_ref[...] = part.reshape(1, 1, 1)


def _sc_gather(tab_ref, idx_ref, out_ref,
               idx_v0, rows_v0, idx_v1, rows_v1,
               idx_v2, rows_v2, idx_v3, rows_v3,
               sem0, sem1, sem2, sem3):
    wid = lax.axis_index("s") * 2 + lax.axis_index("c")
    b_per_w = out_ref.shape[0] // _NW
    base = wid * b_per_w
    bufs = ((idx_v0, rows_v0, sem0), (idx_v1, rows_v1, sem1),
            (idx_v2, rows_v2, sem2), (idx_v3, rows_v3, sem3))

    def quad(jj, carry):
        off = base + (4 * jj) * _CHUNK
        cps = []
        for t, (iv, rv, sm) in enumerate(bufs):
            pltpu.sync_copy(idx_ref.at[pl.ds(off + t * _CHUNK, _CHUNK)], iv)
            cps.append(pltpu.async_copy(tab_ref.at[iv], rv, sm))
        for t, (iv, rv, sm) in enumerate(bufs):
            cps[t].wait()
            pltpu.sync_copy(rv, out_ref.at[pl.ds(off + t * _CHUNK, _CHUNK)])
        return carry

    lax.fori_loop(0, b_per_w // (4 * _CHUNK), quad, 0)


def kernel(inputs, W_shape, W_color):
    b, k, d = inputs.shape
    n = b * k
    grid = n // _T
    flat = inputs.reshape(n, d)

    wsn, wcn, bs, bc, tab3 = pl.pallas_call(
        _prep_body,
        out_shape=[
            jax.ShapeDtypeStruct((_NUM_CODES, _HALF), jnp.float32),
            jax.ShapeDtypeStruct((_COLORS, _HALF), jnp.float32),
            jax.ShapeDtypeStruct((_NUM_CODES, 128), jnp.bfloat16),
            jax.ShapeDtypeStruct((_COLORS, 128), jnp.bfloat16),
            jax.ShapeDtypeStruct((_NUM_CODES, _COLORS, d), jnp.float32),
        ],
    )(W_shape, W_color)
    tab = tab3.reshape(_NUM_CODES * _COLORS, d)

    cat, si, ci, loss_parts = pl.pallas_call(
        _tc_body,
        grid=(grid,),
        in_specs=[
            pl.BlockSpec((_T, d), lambda i: (i, 0)),
            pl.BlockSpec((_NUM_CODES, _HALF), lambda i: (0, 0)),
            pl.BlockSpec((_COLORS, _HALF), lambda i: (0, 0)),
            pl.BlockSpec((_NUM_CODES, 128), lambda i: (0, 0)),
            pl.BlockSpec((_COLORS, 128), lambda i: (0, 0)),
        ],
        out_specs=[
            pl.BlockSpec((_T, 1), lambda i: (i, 0)),
            pl.BlockSpec((_T, 1), lambda i: (i, 0)),
            pl.BlockSpec((_T, 1), lambda i: (i, 0)),
            pl.BlockSpec((1, 1, 1), lambda i: (i, 0, 0)),
        ],
        out_shape=[
            jax.ShapeDtypeStruct((n, 1), jnp.int32),
            jax.ShapeDtypeStruct((n, 1), jnp.int32),
            jax.ShapeDtypeStruct((n, 1), jnp.int32),
            jax.ShapeDtypeStruct((grid, 1, 1), jnp.float32),
        ],
    )(flat, wsn, wcn, bs, bc)

    sc_fn = functools.partial(
        pl.kernel,
        mesh=plsc.VectorSubcoreMesh(core_axis_name="c", subcore_axis_name="s"),
        out_type=jax.ShapeDtypeStruct((n, d), jnp.float32),
        scratch_types=[
            pltpu.VMEM((_CHUNK,), jnp.int32),
            pltpu.VMEM((_CHUNK, d), jnp.float32),
            pltpu.VMEM((_CHUNK,), jnp.int32),
            pltpu.VMEM((_CHUNK, d), jnp.float32),
            pltpu.VMEM((_CHUNK,), jnp.int32),
            pltpu.VMEM((_CHUNK, d), jnp.float32),
            pltpu.VMEM((_CHUNK,), jnp.int32),
            pltpu.VMEM((_CHUNK, d), jnp.float32),
            pltpu.SemaphoreType.DMA,
            pltpu.SemaphoreType.DMA,
            pltpu.SemaphoreType.DMA,
            pltpu.SemaphoreType.DMA,
        ],
    )(_sc_gather)
    q = sc_fn(tab, cat.reshape(n))

    vq_loss = jnp.sum(loss_parts) * (_CC / (n * d))
    return (q.reshape(b, k, d), vq_loss,
            si.reshape(b, k), ci.reshape(b, k))


# speedup vs baseline: 1.0053x; 1.0053x over previous
import functools

import jax
import jax.numpy as jnp
from jax import lax
from jax.experimental import pallas as pl
from jax.experimental.pallas import tpu as pltpu
from jax.experimental.pallas import tpu_sc as plsc

_NUM_CODES = 1024
_HALF = 64
_COLORS = 16
_CC = 0.25

_T = 4096
_NW = 32
_CHUNK = 128


def _rownorm(w):
    return w / jnp.maximum(
        jnp.sqrt(jnp.sum(w * w, axis=1, keepdims=True)), 1e-12)


def _prep_body(ws_ref, wc_ref, wsn_ref, wcn_ref, bs_ref, bc_ref, tab_ref):
    wsn = _rownorm(ws_ref[...])
    wcn = _rownorm(wc_ref[...])
    wsn_ref[...] = wsn
    wcn_ref[...] = wcn

    io_s = jax.lax.broadcasted_iota(jnp.int32, (_NUM_CODES, 1), 0)
    hi = (io_s >> 6).astype(jnp.bfloat16)
    lo = (io_s & 63).astype(jnp.bfloat16)
    bs_ref[...] = jnp.concatenate(
        [hi, lo, jnp.zeros((_NUM_CODES, 126), jnp.bfloat16)], axis=1)

    io_c = jax.lax.broadcasted_iota(jnp.int32, (_COLORS, 1), 0)
    bc_ref[...] = jnp.concatenate(
        [io_c.astype(jnp.bfloat16), jnp.zeros((_COLORS, 127), jnp.bfloat16)],
        axis=1)

    tab_ref[...] = jnp.concatenate([
        jnp.broadcast_to(wsn[:, None, :], (_NUM_CODES, _COLORS, _HALF)),
        jnp.broadcast_to(wcn[None, :, :], (_NUM_CODES, _COLORS, _HALF)),
    ], axis=2)


def _tc_body(z_ref, wsn_ref, wcn_ref, bs_ref, bc_ref,
             cat_ref, si_ref, ci_ref, loss_ref):
    z = z_ref[...]
    zs = z[:, :_HALF]
    zc = z[:, _HALF:]
    nzs2 = jnp.sum(zs * zs, axis=1, keepdims=True)
    nzc2 = jnp.sum(zc * zc, axis=1, keepdims=True)
    nzs = jnp.maximum(jnp.sqrt(nzs2), 1e-12)
    nzc = jnp.maximum(jnp.sqrt(nzc2), 1e-12)
    zsn = zs * (1.0 / nzs)
    zcn = zc * (1.0 / nzc)

    ls = jnp.dot(zsn, wsn_ref[...].T,
                 preferred_element_type=jnp.float32)
    lc = jnp.dot(zcn, wcn_ref[...].T,
                 preferred_element_type=jnp.float32)

    mxs = jnp.max(ls, axis=1, keepdims=True)
    mxc = jnp.max(lc, axis=1, keepdims=True)
    ms = (ls == mxs).astype(jnp.bfloat16)
    mc = (lc == mxc).astype(jnp.bfloat16)

    rs = jnp.dot(ms, bs_ref[...], preferred_element_type=jnp.float32)
    rc = jnp.dot(mc, bc_ref[...], preferred_element_type=jnp.float32)

    si = (rs[:, 0:1] * 64.0 + rs[:, 1:2]).astype(jnp.int32)
    ci = rc[:, 0:1].astype(jnp.int32)
    si_ref[...] = si
    ci_ref[...] = ci
    cat_ref[...] = jnp.minimum(si * _COLORS + ci,
                               _NUM_CODES * _COLORS - 1)

    part = (jnp.sum(nzs2 + nzc2 + 2.0)
            - 2.0 * jnp.sum(mxs * nzs + mxc * nzc))
    loss_ref[...] = part.reshape(1, 1, 1)


def _sc_gather(tab_ref, idx_ref, out_ref,
               idx_v0, rows_v0, idx_v1, rows_v1, sem0, sem1):
    wid = lax.axis_index("s") * 2 + lax.axis_index("c")
    b_per_w = out_ref.shape[0] // _NW
    base = wid * b_per_w

    def pair(jj, carry):
        off0 = base + (2 * jj) * _CHUNK
        off1 = off0 + _CHUNK
        pltpu.sync_copy(idx_ref.at[pl.ds(off0, _CHUNK)], idx_v0)
        cp0 = pltpu.async_copy(tab_ref.at[idx_v0], rows_v0, sem0)
        pltpu.sync_copy(idx_ref.at[pl.ds(off1, _CHUNK)], idx_v1)
        cp1 = pltpu.async_copy(tab_ref.at[idx_v1], rows_v1, sem1)
        cp0.wait()
        pltpu.sync_copy(rows_v0, out_ref.at[pl.ds(off0, _CHUNK)])
        cp1.wait()
        pltpu.sync_copy(rows_v1, out_ref.at[pl.ds(off1, _CHUNK)])
        return carry

    lax.fori_loop(0, b_per_w // (2 * _CHUNK), pair, 0)


def kernel(inputs, W_shape, W_color):
    b, k, d = inputs.shape
    n = b * k
    grid = n // _T
    flat = inputs.reshape(n, d)

    wsn, wcn, bs, bc, tab3 = pl.pallas_call(
        _prep_body,
        out_shape=[
            jax.ShapeDtypeStruct((_NUM_CODES, _HALF), jnp.float32),
            jax.ShapeDtypeStruct((_COLORS, _HALF), jnp.float32),
            jax.ShapeDtypeStruct((_NUM_CODES, 128), jnp.bfloat16),
            jax.ShapeDtypeStruct((_COLORS, 128), jnp.bfloat16),
            jax.ShapeDtypeStruct((_NUM_CODES, _COLORS, d), jnp.float32),
        ],
    )(W_shape, W_color)
    tab = tab3.reshape(_NUM_CODES * _COLORS, d)

    cat, si, ci, loss_parts = pl.pallas_call(
        _tc_body,
        grid=(grid,),
        in_specs=[
            pl.BlockSpec((_T, d), lambda i: (i, 0)),
            pl.BlockSpec((_NUM_CODES, _HALF), lambda i: (0, 0)),
            pl.BlockSpec((_COLORS, _HALF), lambda i: (0, 0)),
            pl.BlockSpec((_NUM_CODES, 128), lambda i: (0, 0)),
            pl.BlockSpec((_COLORS, 128), lambda i: (0, 0)),
        ],
        out_specs=[
            pl.BlockSpec((_T, 1), lambda i: (i, 0)),
            pl.BlockSpec((_T, 1), lambda i: (i, 0)),
            pl.BlockSpec((_T, 1), lambda i: (i, 0)),
            pl.BlockSpec((1, 1, 1), lambda i: (i, 0, 0)),
        ],
        out_shape=[
            jax.ShapeDtypeStruct((n, 1), jnp.int32),
            jax.ShapeDtypeStruct((n, 1), jnp.int32),
            jax.ShapeDtypeStruct((n, 1), jnp.int32),
            jax.ShapeDtypeStruct((grid, 1, 1), jnp.float32),
        ],
    )(flat, wsn, wcn, bs, bc)

    sc_fn = functools.partial(
        pl.kernel,
        mesh=plsc.VectorSubcoreMesh(core_axis_name="c", subcore_axis_name="s"),
        out_type=jax.ShapeDtypeStruct((n, d), jnp.float32),
        scratch_types=[
            pltpu.VMEM((_CHUNK,), jnp.int32),
            pltpu.VMEM((_CHUNK, d), jnp.float32),
            pltpu.VMEM((_CHUNK,), jnp.int32),
            pltpu.VMEM((_CHUNK, d), jnp.float32),
            pltpu.SemaphoreType.DMA,
            pltpu.SemaphoreType.DMA,
        ],
    )(_sc_gather)
    q = sc_fn(tab, cat.reshape(n))

    vq_loss = jnp.sum(loss_parts) * (_CC / (n * d))
    return (q.reshape(b, k, d), vq_loss,
            si.reshape(b, k), ci.reshape(b, k))
